# Initial kernel scaffold; baseline (speedup 1.0000x reference)
import jax
import jax.numpy as jnp
from jax.experimental import pallas as pl


def kernel(encoded_input_sequence, table, w_ih0, w_hh0, b_ih0, b_hh0, w_ih1, w_hh1, b_ih1, b_hh1):
    raise NotImplementedError("write your pallas kernel here")


# speedup vs baseline: 3.4951x; 3.4951x over previous
import functools

import jax
import jax.numpy as jnp
from jax.experimental import pallas as pl
from jax.experimental.pallas import tpu as pltpu

V = 100000
D = 1024
H = 1024
S = 2048


def _gather_body(tok_ref, row_ref, out_ref):
    out_ref[...] = row_ref[...]


def _gather(tokens, table):
    table3 = table.reshape(V, 1, D)
    out = pl.pallas_call(
        _gather_body,
        grid_spec=pltpu.PrefetchScalarGridSpec(
            num_scalar_prefetch=1,
            grid=(S,),
            in_specs=[pl.BlockSpec((1, 1, D), lambda i, tok: (tok[i], 0, 0))],
            out_specs=pl.BlockSpec((1, 1, D), lambda i, tok: (i, 0, 0)),
        ),
        out_shape=jax.ShapeDtypeStruct((S, 1, D), table.dtype),
    )(tokens, table3)
    return out.reshape(S, D)


def _xgates_body(x_ref, w_ref, b_ref, out_ref):
    out_ref[...] = (
        jax.lax.dot_general(
            x_ref[...], w_ref[...], (((1,), (1,)), ((), ())),
            preferred_element_type=jnp.float32,
        )
        + b_ref[...]
    )


def _xgates(x, w_ih, bias):
    n_blk = 4
    blk = 4 * H // n_blk
    return pl.pallas_call(
        _xgates_body,
        grid=(n_blk,),
        in_specs=[
            pl.BlockSpec((S, D), lambda j: (0, 0)),
            pl.BlockSpec((blk, D), lambda j: (j, 0)),
            pl.BlockSpec((1, blk), lambda j: (0, j)),
        ],
        out_specs=pl.BlockSpec((S, blk), lambda j: (0, j)),
        out_shape=jax.ShapeDtypeStruct((S, 4 * H), jnp.float32),
    )(x, w_ih, bias)


def _recurrence_body(xg_ref, whh_ref, out_ref):
    def step(t, carry):
        h, c = carry
        gates = xg_ref[t, :][None, :] + jax.lax.dot_general(
            h, whh_ref[...], (((1,), (1,)), ((), ())),
            preferred_element_type=jnp.float32,
        )
        i = jax.nn.sigmoid(gates[:, 0 * H:1 * H])
        f = jax.nn.sigmoid(gates[:, 1 * H:2 * H])
        g = jnp.tanh(gates[:, 2 * H:3 * H])
        o = jax.nn.sigmoid(gates[:, 3 * H:4 * H])
        c_new = f * c + i * g
        h_new = o * jnp.tanh(c_new)
        out_ref[t, :] = h_new[0]
        return (h_new, c_new)

    h0 = jnp.zeros((1, H), jnp.float32)
    c0 = jnp.zeros((1, H), jnp.float32)
    jax.lax.fori_loop(0, S, step, (h0, c0))


def _recurrence(xgates, w_hh):
    return pl.pallas_call(
        _recurrence_body,
        out_shape=jax.ShapeDtypeStruct((S, H), jnp.float32),
    )(xgates, w_hh)


def _proj_body(h_ref, tab_ref, out_ref):
    out_ref[...] = jax.lax.dot_general(
        h_ref[...], tab_ref[...], (((1,), (1,)), ((), ())),
        preferred_element_type=jnp.float32,
    )


def _projection(h, table):
    blk_v = 1024
    n_blk = pl.cdiv(V, blk_v)
    return pl.pallas_call(
        _proj_body,
        grid=(n_blk,),
        in_specs=[
            pl.BlockSpec((S, H), lambda j: (0, 0)),
            pl.BlockSpec((blk_v, D), lambda j: (j, 0)),
        ],
        out_specs=pl.BlockSpec((S, blk_v), lambda j: (0, j)),
        out_shape=jax.ShapeDtypeStruct((S, V), jnp.float32),
    )(h, table)


@jax.jit
def _run(tokens, table, w_ih0, w_hh0, b0, w_ih1, w_hh1, b1):
    emb = _gather(tokens, table)
    xg0 = _xgates(emb, w_ih0, b0)
    h0_all = _recurrence(xg0, w_hh0)
    xg1 = _xgates(h0_all, w_ih1, b1)
    h1_all = _recurrence(xg1, w_hh1)
    logits = _projection(h1_all, table)
    return logits


def kernel(encoded_input_sequence, table, w_ih0, w_hh0, b_ih0, b_hh0,
           w_ih1, w_hh1, b_ih1, b_hh1):
    tokens = encoded_input_sequence.reshape(-1).astype(jnp.int32)
    b0 = (b_ih0 + b_hh0).reshape(1, -1)
    b1 = (b_ih1 + b_hh1).reshape(1, -1)
    logits = _run(tokens, table, w_ih0, w_hh0, b0, w_ih1, w_hh1, b1)
    return logits[None]


# speedup vs baseline: 7.4515x; 2.1320x over previous
import functools

import jax
import jax.numpy as jnp
from jax.experimental import pallas as pl
from jax.experimental.pallas import tpu as pltpu

V = 100000
D = 1024
H = 1024
S = 2048


_NSEM = 32


def _gather_body(tok_ref, tab_ref, out_ref, sems):
    def copy(i):
        return pltpu.make_async_copy(
            tab_ref.at[pl.ds(tok_ref[i], 1), :],
            out_ref.at[pl.ds(i, 1), :],
            sems.at[jax.lax.rem(i, _NSEM)],
        )

    def issue(i, _):
        @pl.when(i >= _NSEM)
        def _wait():
            copy(i - _NSEM).wait()

        copy(i).start()
        return 0

    jax.lax.fori_loop(0, S, issue, 0)

    def drain(j, _):
        copy(S - _NSEM + j).wait()
        return 0

    jax.lax.fori_loop(0, _NSEM, drain, 0)


def _gather(tokens, table):
    return pl.pallas_call(
        _gather_body,
        grid_spec=pltpu.PrefetchScalarGridSpec(
            num_scalar_prefetch=1,
            grid=(1,),
            in_specs=[pl.BlockSpec(memory_space=pltpu.MemorySpace.HBM)],
            out_specs=pl.BlockSpec(memory_space=pltpu.MemorySpace.HBM),
            scratch_shapes=[pltpu.SemaphoreType.DMA((_NSEM,))],
        ),
        out_shape=jax.ShapeDtypeStruct((S, D), table.dtype),
    )(tokens, table)


def _xgates_body(x_ref, w_ref, b_ref, out_ref):
    out_ref[...] = (
        jax.lax.dot_general(
            x_ref[...], w_ref[...], (((1,), (1,)), ((), ())),
            preferred_element_type=jnp.float32,
        )
        + b_ref[...]
    )


def _xgates(x, w_ih, bias):
    n_blk = 4
    blk = 4 * H // n_blk
    return pl.pallas_call(
        _xgates_body,
        grid=(n_blk,),
        in_specs=[
            pl.BlockSpec((S, D), lambda j: (0, 0)),
            pl.BlockSpec((blk, D), lambda j: (j, 0)),
            pl.BlockSpec((1, blk), lambda j: (0, j)),
        ],
        out_specs=pl.BlockSpec((S, blk), lambda j: (0, j)),
        out_shape=jax.ShapeDtypeStruct((S, 4 * H), jnp.float32),
    )(x, w_ih, bias)


_CHUNK = 256


def _recurrence_body(xg_ref, whh_ref, out_ref, h_ref, c_ref):
    k = pl.program_id(0)

    @pl.when(k == 0)
    def _init():
        h_ref[...] = jnp.zeros_like(h_ref)
        c_ref[...] = jnp.zeros_like(c_ref)

    def step(t, carry):
        h, c = carry
        hb = h.astype(jnp.bfloat16)

        def gate_dot(idx):
            return jax.lax.dot_general(
                hb, whh_ref[:, idx * H:(idx + 1) * H],
                (((1,), (0,)), ((), ())),
                preferred_element_type=jnp.float32,
            ) + xg_ref[t, idx * H:(idx + 1) * H][None, :]

        i = jax.nn.sigmoid(gate_dot(0))
        f = jax.nn.sigmoid(gate_dot(1))
        g = jnp.tanh(gate_dot(2))
        o = jax.nn.sigmoid(gate_dot(3))
        c_new = f * c + i * g
        h_new = o * jnp.tanh(c_new)
        out_ref[t, :] = h_new[0]
        return (h_new, c_new)

    carry = jax.lax.fori_loop(0, _CHUNK, step, (h_ref[...], c_ref[...]),
                              unroll=4)
    h_ref[...], c_ref[...] = carry


def _recurrence(xgates, w_hh):
    return pl.pallas_call(
        _recurrence_body,
        grid=(S // _CHUNK,),
        in_specs=[
            pl.BlockSpec((_CHUNK, 4 * H), lambda k: (k, 0)),
            pl.BlockSpec((H, 4 * H), lambda k: (0, 0)),
        ],
        out_specs=pl.BlockSpec((_CHUNK, H), lambda k: (k, 0)),
        out_shape=jax.ShapeDtypeStruct((S, H), jnp.float32),
        scratch_shapes=[
            pltpu.VMEM((1, H), jnp.float32),
            pltpu.VMEM((1, H), jnp.float32),
        ],
    )(xgates, w_hh.T.astype(jnp.bfloat16))


def _proj_body(h_ref, tab_ref, out_ref):
    out_ref[...] = jax.lax.dot_general(
        tab_ref[...], h_ref[...], (((1,), (1,)), ((), ())),
        preferred_element_type=jnp.float32,
    )


def _projection(h, table):
    blk_v = 1024
    n_blk = pl.cdiv(V, blk_v)
    return pl.pallas_call(
        _proj_body,
        grid=(n_blk,),
        in_specs=[
            pl.BlockSpec((S, H), lambda j: (0, 0)),
            pl.BlockSpec((blk_v, D), lambda j: (j, 0)),
        ],
        out_specs=pl.BlockSpec((blk_v, S), lambda j: (j, 0)),
        out_shape=jax.ShapeDtypeStruct((V, S), jnp.float32),
    )(h, table)


@jax.jit
def _run(tokens, table, w_ih0, w_hh0, b0, w_ih1, w_hh1, b1):
    emb = _gather(tokens, table)
    xg0 = _xgates(emb, w_ih0, b0)
    h0_all = _recurrence(xg0, w_hh0)
    xg1 = _xgates(h0_all, w_ih1, b1)
    h1_all = _recurrence(xg1, w_hh1)
    logits_t = _projection(h1_all, table)
    return logits_t


def kernel(encoded_input_sequence, table, w_ih0, w_hh0, b_ih0, b_hh0,
           w_ih1, w_hh1, b_ih1, b_hh1):
    tokens = encoded_input_sequence.reshape(-1).astype(jnp.int32)
    b0 = (b_ih0 + b_hh0).reshape(1, -1)
    b1 = (b_ih1 + b_hh1).reshape(1, -1)
    logits_t = _run(tokens, table, w_ih0, w_hh0, b0, w_ih1, w_hh1, b1)
    return jnp.swapaxes(logits_t, 0, 1)[None]


# speedup vs baseline: 7.7148x; 1.0353x over previous
import jax
import jax.numpy as jnp
from jax.experimental import pallas as pl
from jax.experimental.pallas import tpu as pltpu

V = 100000
D = 1024
H = 1024
S = 2048

_CHUNK = 256


def _lstm_steps(xg_ref, whh_ref, out_ref, h_ref, c_ref):
    def step(t, carry):
        h, c = carry
        hb = h.astype(jnp.bfloat16)

        def gate_dot(idx):
            return jax.lax.dot_general(
                hb, whh_ref[:, idx * H:(idx + 1) * H],
                (((1,), (0,)), ((), ())),
                preferred_element_type=jnp.float32,
            ) + xg_ref[t, idx * H:(idx + 1) * H][None, :]

        i = jax.nn.sigmoid(gate_dot(0))
        f = jax.nn.sigmoid(gate_dot(1))
        g = jnp.tanh(gate_dot(2))
        o = jax.nn.sigmoid(gate_dot(3))
        c_new = f * c + i * g
        h_new = o * jnp.tanh(c_new)
        out_ref[t, :] = h_new[0]
        return (h_new, c_new)

    carry = jax.lax.fori_loop(0, _CHUNK, step, (h_ref[...], c_ref[...]),
                              unroll=4)
    h_ref[...], c_ref[...] = carry


def _layer0_body(tok_ref, tab_ref, wih_ref, b_ref, whh_ref, out_ref,
                 emb_ref, xg_ref, h_ref, c_ref, sems):
    k = pl.program_id(0)
    nk = pl.num_programs(0)

    def row_copy(kk, i):
        return pltpu.make_async_copy(
            tab_ref.at[pl.ds(tok_ref[kk * _CHUNK + i], 1), :],
            emb_ref.at[kk % 2, pl.ds(i, 1), :],
            sems.at[kk % 2],
        )

    def issue_chunk(kk):
        def issue(i, _):
            row_copy(kk, i).start()
            return 0
        jax.lax.fori_loop(0, _CHUNK, issue, 0)

    @pl.when(k == 0)
    def _first():
        h_ref[...] = jnp.zeros_like(h_ref)
        c_ref[...] = jnp.zeros_like(c_ref)
        issue_chunk(0)

    @pl.when(k + 1 < nk)
    def _ahead():
        issue_chunk(k + 1)

    def wait_chunk(i, _):
        row_copy(k, i).wait()
        return 0

    jax.lax.fori_loop(0, _CHUNK, wait_chunk, 0)

    xg_ref[...] = jax.lax.dot_general(
        emb_ref[k % 2], wih_ref[...], (((1,), (0,)), ((), ())),
        preferred_element_type=jnp.float32,
    ) + b_ref[...]

    _lstm_steps(xg_ref, whh_ref, out_ref, h_ref, c_ref)


def _layer0(tokens, table, w_ih, bias, w_hh):
    return pl.pallas_call(
        _layer0_body,
        grid_spec=pltpu.PrefetchScalarGridSpec(
            num_scalar_prefetch=1,
            grid=(S // _CHUNK,),
            in_specs=[
                pl.BlockSpec(memory_space=pltpu.MemorySpace.HBM),
                pl.BlockSpec((D, 4 * H), lambda k, tok: (0, 0)),
                pl.BlockSpec((1, 4 * H), lambda k, tok: (0, 0)),
                pl.BlockSpec((H, 4 * H), lambda k, tok: (0, 0)),
            ],
            out_specs=pl.BlockSpec((_CHUNK, H), lambda k, tok: (k, 0)),
            scratch_shapes=[
                pltpu.VMEM((2, _CHUNK, D), jnp.float32),
                pltpu.VMEM((_CHUNK, 4 * H), jnp.float32),
                pltpu.VMEM((1, H), jnp.float32),
                pltpu.VMEM((1, H), jnp.float32),
                pltpu.SemaphoreType.DMA((2,)),
            ],
        ),
        out_shape=jax.ShapeDtypeStruct((S, H), jnp.float32),
    )(tokens, table, w_ih.T, bias, w_hh.T.astype(jnp.bfloat16))


def _layer1_body(x_ref, wih_ref, b_ref, whh_ref, out_ref,
                 xg_ref, h_ref, c_ref):
    k = pl.program_id(0)

    @pl.when(k == 0)
    def _first():
        h_ref[...] = jnp.zeros_like(h_ref)
        c_ref[...] = jnp.zeros_like(c_ref)

    xg_ref[...] = jax.lax.dot_general(
        x_ref[...], wih_ref[...], (((1,), (0,)), ((), ())),
        preferred_element_type=jnp.float32,
    ) + b_ref[...]

    _lstm_steps(xg_ref, whh_ref, out_ref, h_ref, c_ref)


def _layer1(x, w_ih, bias, w_hh):
    return pl.pallas_call(
        _layer1_body,
        grid=(S // _CHUNK,),
        in_specs=[
            pl.BlockSpec((_CHUNK, H), lambda k: (k, 0)),
            pl.BlockSpec((H, 4 * H), lambda k: (0, 0)),
            pl.BlockSpec((1, 4 * H), lambda k: (0, 0)),
            pl.BlockSpec((H, 4 * H), lambda k: (0, 0)),
        ],
        out_specs=pl.BlockSpec((_CHUNK, H), lambda k: (k, 0)),
        out_shape=jax.ShapeDtypeStruct((S, H), jnp.float32),
        scratch_shapes=[
            pltpu.VMEM((_CHUNK, 4 * H), jnp.float32),
            pltpu.VMEM((1, H), jnp.float32),
            pltpu.VMEM((1, H), jnp.float32),
        ],
    )(x, w_ih.T, bias, w_hh.T.astype(jnp.bfloat16))


def _proj_body(h_ref, tab_ref, out_ref):
    out_ref[...] = jax.lax.dot_general(
        tab_ref[...], h_ref[...], (((1,), (1,)), ((), ())),
        preferred_element_type=jnp.float32,
    )


def _projection(h, table):
    blk_v = 1024
    n_blk = pl.cdiv(V, blk_v)
    return pl.pallas_call(
        _proj_body,
        grid=(n_blk,),
        in_specs=[
            pl.BlockSpec((S, H), lambda j: (0, 0)),
            pl.BlockSpec((blk_v, D), lambda j: (j, 0)),
        ],
        out_specs=pl.BlockSpec((blk_v, S), lambda j: (j, 0)),
        out_shape=jax.ShapeDtypeStruct((V, S), jnp.float32),
    )(h, table)


@jax.jit
def _run(tokens, table, w_ih0, w_hh0, b0, w_ih1, w_hh1, b1):
    h0_all = _layer0(tokens, table, w_ih0, b0, w_hh0)
    h1_all = _layer1(h0_all, w_ih1, b1, w_hh1)
    return _projection(h1_all, table)


def kernel(encoded_input_sequence, table, w_ih0, w_hh0, b_ih0, b_hh0,
           w_ih1, w_hh1, b_ih1, b_hh1):
    tokens = encoded_input_sequence.reshape(-1).astype(jnp.int32)
    b0 = (b_ih0 + b_hh0).reshape(1, -1)
    b1 = (b_ih1 + b_hh1).reshape(1, -1)
    logits_t = _run(tokens, table, w_ih0, w_hh0, b0, w_ih1, w_hh1, b1)
    return jnp.swapaxes(logits_t, 0, 1)[None]
